# Initial kernel scaffold; baseline (speedup 1.0000x reference)
import jax
import jax.numpy as jnp
from jax.experimental import pallas as pl


def kernel(xyz, pe):
    raise NotImplementedError("write your pallas kernel here")


# speedup vs baseline: 2.9917x; 2.9917x over previous
import functools

import jax
import jax.numpy as jnp
from jax import lax
from jax.experimental import pallas as pl
from jax.experimental.pallas import tpu as pltpu
from jax.experimental.pallas import tpu_sc as plsc

D_MODEL = 512
TABLE_ROWS = 10000
P1 = 73856093 % TABLE_ROWS
P2 = 19349663 % TABLE_ROWS
P3 = 83492791 % TABLE_ROWS
CHUNK = 64
LANES = 16


def _build_sc_call(total, nw):
    b_per_w = total // nw
    nch = b_per_w // CHUNK
    assert b_per_w % CHUNK == 0 and nch % 2 == 0
    grp = b_per_w // LANES
    mesh = plsc.VectorSubcoreMesh(core_axis_name="c", subcore_axis_name="s")
    nc = mesh.num_cores

    @functools.partial(
        pl.kernel,
        out_type=jax.ShapeDtypeStruct((total, D_MODEL), jnp.float32),
        mesh=mesh,
        scratch_types=[
            pltpu.VMEM((b_per_w,), jnp.float32),
            pltpu.VMEM((b_per_w,), jnp.float32),
            pltpu.VMEM((b_per_w,), jnp.float32),
            pltpu.VMEM((b_per_w,), jnp.int32),
            pltpu.VMEM((2, CHUNK, D_MODEL), jnp.float32),
            pltpu.SemaphoreType.DMA,
            pltpu.SemaphoreType.DMA,
        ],
    )
    def sc_kernel(x_hbm, y_hbm, z_hbm, pe_hbm, out_hbm, x_v, y_v, z_v, idx_v, rows_v, sem0, sem1):
        wid = lax.axis_index("s") * nc + lax.axis_index("c")
        base_pt = wid * b_per_w

        pltpu.sync_copy(x_hbm.at[pl.ds(base_pt, b_per_w)], x_v)
        pltpu.sync_copy(y_hbm.at[pl.ds(base_pt, b_per_w)], y_v)
        pltpu.sync_copy(z_hbm.at[pl.ds(base_pt, b_per_w)], z_v)

        m_i32 = jnp.int32(TABLE_ROWS)
        comps = (x_v, y_v, z_v)

        def hash_body(_, pt):
            def term(comp, mult):
                v = comps[comp][pl.ds(pt, LANES)]
                a = (v * 1000.0).astype(jnp.int32)
                r = lax.rem(a, m_i32)
                r = jnp.where(r < 0, r + m_i32, r)
                return r * jnp.int32(mult)

            h = term(0, P1) + term(1, P2) + term(2, P3)
            idx_v[pl.ds(pt, LANES)] = lax.rem(h, m_i32)
            return pt + jnp.int32(LANES)

        lax.fori_loop(0, grp, hash_body, jnp.int32(0), unroll=2)

        sems = (sem0, sem1)

        def start(t, b):
            pltpu.async_copy(
                pe_hbm.at[idx_v.at[pl.ds(t * CHUNK, CHUNK)]],
                rows_v.at[jnp.int32(b)],
                sems[b],
            )

        def wait(b):
            pltpu.make_async_copy(
                pe_hbm.at[idx_v.at[pl.ds(0, CHUNK)]], rows_v.at[jnp.int32(b)], sems[b]
            ).wait()

        start(0, 0)
        start(1, 1)

        def pipe_body(_, t0):
            for b in range(2):
                t = t0 + jnp.int32(b)
                wait(b)
                pltpu.sync_copy(
                    rows_v.at[jnp.int32(b)], out_hbm.at[pl.ds(base_pt + t * CHUNK, CHUNK)]
                )

                @pl.when(t + 2 < nch)
                def _():
                    start(t + jnp.int32(2), b)

            return t0 + jnp.int32(2)

        lax.fori_loop(0, nch // 2, pipe_body, jnp.int32(0))

    return sc_kernel


def kernel(xyz, pe):
    b, n, _ = xyz.shape
    total = b * n
    info = plsc.get_sparse_core_info()
    nw = info.num_cores * info.num_subcores
    sc_call = _build_sc_call(total, nw)
    flat = xyz.reshape(total, 3)
    out = sc_call(flat[:, 0], flat[:, 1], flat[:, 2], pe)
    return out.reshape(b, n, D_MODEL)


# speedup vs baseline: 3.0017x; 1.0034x over previous
import functools

import jax
import jax.numpy as jnp
from jax import lax
from jax.experimental import pallas as pl
from jax.experimental.pallas import tpu as pltpu
from jax.experimental.pallas import tpu_sc as plsc

D_MODEL = 512
TABLE_ROWS = 10000
P1 = 73856093 % TABLE_ROWS
P2 = 19349663 % TABLE_ROWS
P3 = 83492791 % TABLE_ROWS
CHUNK = 32
NBUF = 4
LANES = 16


def _build_sc_call(total, nw):
    b_per_w = total // nw
    nch = b_per_w // CHUNK
    assert b_per_w % CHUNK == 0 and nch % NBUF == 0 and nch >= 2 * NBUF
    grp = b_per_w // LANES
    mesh = plsc.VectorSubcoreMesh(core_axis_name="c", subcore_axis_name="s")
    nc = mesh.num_cores

    @functools.partial(
        pl.kernel,
        out_type=jax.ShapeDtypeStruct((total, D_MODEL), jnp.float32),
        mesh=mesh,
        scratch_types=[
            pltpu.VMEM((b_per_w,), jnp.float32),
            pltpu.VMEM((b_per_w,), jnp.float32),
            pltpu.VMEM((b_per_w,), jnp.float32),
            pltpu.VMEM((b_per_w,), jnp.int32),
            pltpu.VMEM((NBUF, CHUNK, D_MODEL), jnp.float32),
            pltpu.SemaphoreType.DMA,
            pltpu.SemaphoreType.DMA,
            pltpu.SemaphoreType.DMA,
            pltpu.SemaphoreType.DMA,
            pltpu.SemaphoreType.DMA,
            pltpu.SemaphoreType.DMA,
            pltpu.SemaphoreType.DMA,
            pltpu.SemaphoreType.DMA,
        ],
    )
    def sc_kernel(x_hbm, y_hbm, z_hbm, pe_hbm, out_hbm, x_v, y_v, z_v, idx_v, rows_v,
                  g0, g1, g2, g3, w0, w1, w2, w3):
        gsems = (g0, g1, g2, g3)
        wsems = (w0, w1, w2, w3)
        wid = lax.axis_index("s") * nc + lax.axis_index("c")
        base_pt = wid * b_per_w

        pltpu.sync_copy(x_hbm.at[pl.ds(base_pt, b_per_w)], x_v)
        pltpu.sync_copy(y_hbm.at[pl.ds(base_pt, b_per_w)], y_v)
        pltpu.sync_copy(z_hbm.at[pl.ds(base_pt, b_per_w)], z_v)

        m_i32 = jnp.int32(TABLE_ROWS)
        comps = (x_v, y_v, z_v)

        def hash_body(_, pt):
            def term(comp, mult):
                v = comps[comp][pl.ds(pt, LANES)]
                a = (v * 1000.0).astype(jnp.int32)
                r = lax.rem(a, m_i32)
                r = jnp.where(r < 0, r + m_i32, r)
                return r * jnp.int32(mult)

            h = term(0, P1) + term(1, P2) + term(2, P3)
            idx_v[pl.ds(pt, LANES)] = lax.rem(h, m_i32)
            return pt + jnp.int32(LANES)

        lax.fori_loop(0, grp, hash_body, jnp.int32(0), unroll=2)

        def start_g(t, b):
            pltpu.async_copy(
                pe_hbm.at[idx_v.at[pl.ds(t * CHUNK, CHUNK)]],
                rows_v.at[jnp.int32(b)],
                gsems[b],
            )

        def wait_g(b):
            pltpu.make_async_copy(
                pe_hbm.at[idx_v.at[pl.ds(0, CHUNK)]], rows_v.at[jnp.int32(b)], gsems[b]
            ).wait()

        def start_w(t, b):
            pltpu.async_copy(
                rows_v.at[jnp.int32(b)],
                out_hbm.at[pl.ds(base_pt + t * CHUNK, CHUNK)],
                wsems[b],
            )

        def wait_w(b):
            pltpu.make_async_copy(
                rows_v.at[jnp.int32(b)], out_hbm.at[pl.ds(0, CHUNK)], wsems[b]
            ).wait()

        start_g(0, 0)
        start_g(1, 1)

        def pipe_body(_, t0):
            for b in range(NBUF):
                t = t0 + jnp.int32(b)
                wait_g(b)
                start_w(t, b)
                b2 = (b + 2) % NBUF
                t2 = t + jnp.int32(2)

                @pl.when((t >= 2) & (t2 < nch))
                def _():
                    wait_w(b2)

                @pl.when(t2 < nch)
                def _():
                    start_g(t2, b2)

            return t0 + jnp.int32(NBUF)

        lax.fori_loop(0, nch // NBUF, pipe_body, jnp.int32(0))
        for b in range(NBUF):
            wait_w(b)

    return sc_kernel


def kernel(xyz, pe):
    b, n, _ = xyz.shape
    total = b * n
    info = plsc.get_sparse_core_info()
    nw = info.num_cores * info.num_subcores
    sc_call = _build_sc_call(total, nw)
    flat = xyz.reshape(total, 3)
    out = sc_call(flat[:, 0], flat[:, 1], flat[:, 2], pe)
    return out.reshape(b, n, D_MODEL)


# speedup vs baseline: 3.0200x; 1.0061x over previous
import functools

import jax
import jax.numpy as jnp
from jax import lax
from jax.experimental import pallas as pl
from jax.experimental.pallas import tpu as pltpu
from jax.experimental.pallas import tpu_sc as plsc

D_MODEL = 512
TABLE_ROWS = 10000
P1 = 73856093 % TABLE_ROWS
P2 = 19349663 % TABLE_ROWS
P3 = 83492791 % TABLE_ROWS
CHUNK = 16
NBUF = 8
DEPTH = 4
LANES = 16


def _build_sc_call(total, nw):
    b_per_w = total // nw
    nch = b_per_w // CHUNK
    assert b_per_w % CHUNK == 0 and nch % NBUF == 0 and nch >= 2 * NBUF
    grp = b_per_w // LANES
    mesh = plsc.VectorSubcoreMesh(core_axis_name="c", subcore_axis_name="s")
    nc = mesh.num_cores

    @functools.partial(
        pl.kernel,
        out_type=jax.ShapeDtypeStruct((total, D_MODEL), jnp.float32),
        mesh=mesh,
        scratch_types=[
            pltpu.VMEM((b_per_w,), jnp.float32),
            pltpu.VMEM((b_per_w,), jnp.float32),
            pltpu.VMEM((b_per_w,), jnp.float32),
            pltpu.VMEM((b_per_w,), jnp.int32),
            pltpu.VMEM((NBUF, CHUNK, D_MODEL), jnp.float32),
            *([pltpu.SemaphoreType.DMA] * (2 * NBUF)),
        ],
    )
    def sc_kernel(x_hbm, y_hbm, z_hbm, pe_hbm, out_hbm, x_v, y_v, z_v, idx_v, rows_v,
                  *sems):
        gsems = sems[:NBUF]
        wsems = sems[NBUF:]
        wid = lax.axis_index("s") * nc + lax.axis_index("c")
        base_pt = wid * b_per_w

        pltpu.sync_copy(x_hbm.at[pl.ds(base_pt, b_per_w)], x_v)
        pltpu.sync_copy(y_hbm.at[pl.ds(base_pt, b_per_w)], y_v)
        pltpu.sync_copy(z_hbm.at[pl.ds(base_pt, b_per_w)], z_v)

        m_i32 = jnp.int32(TABLE_ROWS)
        comps = (x_v, y_v, z_v)

        def hash_body(_, pt):
            def term(comp, mult):
                v = comps[comp][pl.ds(pt, LANES)]
                a = (v * 1000.0).astype(jnp.int32)
                r = lax.rem(a, m_i32)
                r = jnp.where(r < 0, r + m_i32, r)
                return r * jnp.int32(mult)

            h = term(0, P1) + term(1, P2) + term(2, P3)
            idx_v[pl.ds(pt, LANES)] = lax.rem(h, m_i32)
            return pt + jnp.int32(LANES)

        lax.fori_loop(0, grp, hash_body, jnp.int32(0), unroll=2)

        def start_g(t, b):
            pltpu.async_copy(
                pe_hbm.at[idx_v.at[pl.ds(t * CHUNK, CHUNK)]],
                rows_v.at[jnp.int32(b)],
                gsems[b],
            )

        def wait_g(b):
            pltpu.make_async_copy(
                pe_hbm.at[idx_v.at[pl.ds(0, CHUNK)]], rows_v.at[jnp.int32(b)], gsems[b]
            ).wait()

        def start_w(t, b):
            pltpu.async_copy(
                rows_v.at[jnp.int32(b)],
                out_hbm.at[pl.ds(base_pt + t * CHUNK, CHUNK)],
                wsems[b],
            )

        def wait_w(b):
            pltpu.make_async_copy(
                rows_v.at[jnp.int32(b)], out_hbm.at[pl.ds(0, CHUNK)], wsems[b]
            ).wait()

        for t in range(DEPTH):
            start_g(t, t)

        def pipe_body(_, t0):
            for b in range(NBUF):
                t = t0 + jnp.int32(b)
                wait_g(b)
                start_w(t, b)
                bd = (b + DEPTH) % NBUF
                td = t + jnp.int32(DEPTH)

                @pl.when((t >= NBUF - DEPTH) & (td < nch))
                def _():
                    wait_w(bd)

                @pl.when(td < nch)
                def _():
                    start_g(td, bd)

            return t0 + jnp.int32(NBUF)

        lax.fori_loop(0, nch // NBUF, pipe_body, jnp.int32(0))
        for b in range(NBUF):
            wait_w(b)

    return sc_kernel


def kernel(xyz, pe):
    b, n, _ = xyz.shape
    total = b * n
    info = plsc.get_sparse_core_info()
    nw = info.num_cores * info.num_subcores
    sc_call = _build_sc_call(total, nw)
    flat = xyz.reshape(total, 3)
    out = sc_call(flat[:, 0], flat[:, 1], flat[:, 2], pe)
    return out.reshape(b, n, D_MODEL)


# speedup vs baseline: 3.3339x; 1.1039x over previous
import functools

import jax
import jax.numpy as jnp
from jax import lax
from jax.experimental import pallas as pl
from jax.experimental.pallas import tpu as pltpu
from jax.experimental.pallas import tpu_sc as plsc

D_MODEL = 512
TABLE_ROWS = 10000
P1 = 73856093 % TABLE_ROWS
P2 = 19349663 % TABLE_ROWS
P3 = 83492791 % TABLE_ROWS
CHUNK = 16
NBUF = 8
DEPTH = 4
LANES = 16


def _build_sc_call(total, nw):
    b_per_w = total // nw
    nch = b_per_w // CHUNK
    assert b_per_w % CHUNK == 0 and nch % NBUF == 0 and nch >= 2 * NBUF
    grp = b_per_w // LANES
    mesh = plsc.VectorSubcoreMesh(core_axis_name="c", subcore_axis_name="s")
    nc = mesh.num_cores

    @functools.partial(
        pl.kernel,
        out_type=jax.ShapeDtypeStruct((total, D_MODEL), jnp.float32),
        mesh=mesh,
        scratch_types=[
            pltpu.VMEM((b_per_w,), jnp.float32),
            pltpu.VMEM((b_per_w,), jnp.float32),
            pltpu.VMEM((b_per_w,), jnp.float32),
            pltpu.VMEM((b_per_w,), jnp.int32),
            pltpu.VMEM((NBUF, CHUNK, D_MODEL), jnp.float32),
            *([pltpu.SemaphoreType.DMA] * (2 * NBUF)),
        ],
    )
    def sc_kernel(x_hbm, y_hbm, z_hbm, pe_hbm, out_hbm, x_v, y_v, z_v, idx_v, rows_v,
                  *sems):
        gsems = sems[:NBUF]
        wsems = sems[NBUF:]
        wid = lax.axis_index("s") * nc + lax.axis_index("c")
        base_pt = wid * b_per_w

        cp_x = pltpu.async_copy(x_hbm.at[pl.ds(base_pt, b_per_w)], x_v, sems[0])
        cp_y = pltpu.async_copy(y_hbm.at[pl.ds(base_pt, b_per_w)], y_v, sems[1])
        cp_z = pltpu.async_copy(z_hbm.at[pl.ds(base_pt, b_per_w)], z_v, sems[2])
        cp_x.wait()
        cp_y.wait()
        cp_z.wait()

        m_i32 = jnp.int32(TABLE_ROWS)
        comps = (x_v, y_v, z_v)

        def hash_chunk(t):
            pt = t * jnp.int32(CHUNK)

            def term(comp, mult):
                v = comps[comp][pl.ds(pt, LANES)]
                a = (v * 1000.0).astype(jnp.int32)
                r = lax.rem(a, m_i32)
                r = jnp.where(r < 0, r + m_i32, r)
                return r * jnp.int32(mult)

            h = term(0, P1) + term(1, P2) + term(2, P3)
            idx_v[pl.ds(pt, LANES)] = lax.rem(h, m_i32)

        def start_g(t, b):
            pltpu.async_copy(
                pe_hbm.at[idx_v.at[pl.ds(t * CHUNK, CHUNK)]],
                rows_v.at[jnp.int32(b)],
                gsems[b],
            )

        def wait_g(b):
            pltpu.make_async_copy(
                pe_hbm.at[idx_v.at[pl.ds(0, CHUNK)]], rows_v.at[jnp.int32(b)], gsems[b]
            ).wait()

        def start_w(t, b):
            pltpu.async_copy(
                rows_v.at[jnp.int32(b)],
                out_hbm.at[pl.ds(base_pt + t * CHUNK, CHUNK)],
                wsems[b],
            )

        def wait_w(b):
            pltpu.make_async_copy(
                rows_v.at[jnp.int32(b)], out_hbm.at[pl.ds(0, CHUNK)], wsems[b]
            ).wait()

        for t in range(DEPTH):
            hash_chunk(jnp.int32(t))
            start_g(t, t)

        def pipe_body(_, t0):
            for b in range(NBUF):
                t = t0 + jnp.int32(b)
                wait_g(b)
                start_w(t, b)
                bd = (b + DEPTH) % NBUF
                td = t + jnp.int32(DEPTH)

                @pl.when((t >= NBUF - DEPTH) & (td < nch))
                def _():
                    wait_w(bd)

                @pl.when(td < nch)
                def _():
                    hash_chunk(td)
                    start_g(td, bd)

            return t0 + jnp.int32(NBUF)

        lax.fori_loop(0, nch // NBUF, pipe_body, jnp.int32(0))
        for b in range(NBUF):
            wait_w(b)

    return sc_kernel


def kernel(xyz, pe):
    b, n, _ = xyz.shape
    total = b * n
    info = plsc.get_sparse_core_info()
    nw = info.num_cores * info.num_subcores
    sc_call = _build_sc_call(total, nw)
    flat = xyz.reshape(total, 3)
    out = sc_call(flat[:, 0], flat[:, 1], flat[:, 2], pe)
    return out.reshape(b, n, D_MODEL)


# speedup vs baseline: 3.3908x; 1.0171x over previous
import functools

import jax
import jax.numpy as jnp
from jax import lax
from jax.experimental import pallas as pl
from jax.experimental.pallas import tpu as pltpu
from jax.experimental.pallas import tpu_sc as plsc

D_MODEL = 512
TABLE_ROWS = 10000
P1 = 73856093 % TABLE_ROWS
P2 = 19349663 % TABLE_ROWS
P3 = 83492791 % TABLE_ROWS
CHUNK = 16
NBUF = 8
DEPTH = 5
LANES = 16


def _build_sc_call(total, nw):
    b_per_w = total // nw
    nch = b_per_w // CHUNK
    assert b_per_w % CHUNK == 0 and nch % NBUF == 0 and nch >= 2 * NBUF
    grp = b_per_w // LANES
    mesh = plsc.VectorSubcoreMesh(core_axis_name="c", subcore_axis_name="s")
    nc = mesh.num_cores

    @functools.partial(
        pl.kernel,
        out_type=jax.ShapeDtypeStruct((total, D_MODEL), jnp.float32),
        mesh=mesh,
        scratch_types=[
            pltpu.VMEM((b_per_w,), jnp.float32),
            pltpu.VMEM((b_per_w,), jnp.float32),
            pltpu.VMEM((b_per_w,), jnp.float32),
            pltpu.VMEM((b_per_w,), jnp.int32),
            pltpu.VMEM((NBUF, CHUNK, D_MODEL), jnp.float32),
            *([pltpu.SemaphoreType.DMA] * (2 * NBUF)),
        ],
    )
    def sc_kernel(x_hbm, y_hbm, z_hbm, pe_hbm, out_hbm, x_v, y_v, z_v, idx_v, rows_v,
                  *sems):
        gsems = sems[:NBUF]
        wsems = sems[NBUF:]
        wid = lax.axis_index("s") * nc + lax.axis_index("c")
        base_pt = wid * b_per_w

        cp_x = pltpu.async_copy(x_hbm.at[pl.ds(base_pt, b_per_w)], x_v, sems[0])
        cp_y = pltpu.async_copy(y_hbm.at[pl.ds(base_pt, b_per_w)], y_v, sems[1])
        cp_z = pltpu.async_copy(z_hbm.at[pl.ds(base_pt, b_per_w)], z_v, sems[2])
        cp_x.wait()
        cp_y.wait()
        cp_z.wait()

        m_i32 = jnp.int32(TABLE_ROWS)
        comps = (x_v, y_v, z_v)

        def hash_chunk(t):
            pt = t * jnp.int32(CHUNK)

            def term(comp, mult):
                v = comps[comp][pl.ds(pt, LANES)]
                a = (v * 1000.0).astype(jnp.int32)
                r = lax.rem(a, m_i32)
                r = jnp.where(r < 0, r + m_i32, r)
                return r * jnp.int32(mult)

            h = term(0, P1) + term(1, P2) + term(2, P3)
            idx_v[pl.ds(pt, LANES)] = lax.rem(h, m_i32)

        def start_g(t, b):
            pltpu.async_copy(
                pe_hbm.at[idx_v.at[pl.ds(t * CHUNK, CHUNK)]],
                rows_v.at[jnp.int32(b)],
                gsems[b],
            )

        def wait_g(b):
            pltpu.make_async_copy(
                pe_hbm.at[idx_v.at[pl.ds(0, CHUNK)]], rows_v.at[jnp.int32(b)], gsems[b]
            ).wait()

        def start_w(t, b):
            pltpu.async_copy(
                rows_v.at[jnp.int32(b)],
                out_hbm.at[pl.ds(base_pt + t * CHUNK, CHUNK)],
                wsems[b],
            )

        def wait_w(b):
            pltpu.make_async_copy(
                rows_v.at[jnp.int32(b)], out_hbm.at[pl.ds(0, CHUNK)], wsems[b]
            ).wait()

        for t in range(DEPTH):
            hash_chunk(jnp.int32(t))
            start_g(t, t)

        def pipe_body(_, t0):
            for b in range(NBUF):
                t = t0 + jnp.int32(b)
                wait_g(b)
                start_w(t, b)
                bd = (b + DEPTH) % NBUF
                td = t + jnp.int32(DEPTH)

                @pl.when((t >= NBUF - DEPTH) & (td < nch))
                def _():
                    wait_w(bd)

                @pl.when(td < nch)
                def _():
                    hash_chunk(td)
                    start_g(td, bd)

            return t0 + jnp.int32(NBUF)

        lax.fori_loop(0, nch // NBUF, pipe_body, jnp.int32(0))
        for b in range(NBUF):
            wait_w(b)

    return sc_kernel


def kernel(xyz, pe):
    b, n, _ = xyz.shape
    total = b * n
    info = plsc.get_sparse_core_info()
    nw = info.num_cores * info.num_subcores
    sc_call = _build_sc_call(total, nw)
    flat = xyz.reshape(total, 3)
    out = sc_call(flat[:, 0], flat[:, 1], flat[:, 2], pe)
    return out.reshape(b, n, D_MODEL)
